# Initial kernel scaffold; baseline (speedup 1.0000x reference)
import jax
import jax.numpy as jnp
from jax.experimental import pallas as pl


def kernel(predicate, input, Wp, bp, We, be):
    raise NotImplementedError("write your pallas kernel here")


# speedup vs baseline: 1.9904x; 1.9904x over previous
import functools

import jax
import jax.numpy as jnp
from jax.experimental import pallas as pl
from jax.experimental.pallas import tpu as pltpu

_E = 8
_D = 2048
_NTOK = 4096

_BM = 1024
_BN = 512
_BK = 512


def _route_kernel(pred_ref, wp_ref, bp_ref, be_ref,
                  t0_ref, t1_ref, w0_ref, w1_ref, bc_ref):
    pred = jnp.dot(pred_ref[...], wp_ref[...],
                   preferred_element_type=jnp.float32) + bp_ref[...]
    iota = jax.lax.broadcasted_iota(jnp.int32, pred.shape, 1)
    big = jnp.int32(_E + 1)
    v0 = jnp.max(pred)
    t0 = jnp.min(jnp.where(pred == v0, iota, big))
    m0 = iota == t0
    pred1 = jnp.where(m0, -jnp.inf, pred)
    v1 = jnp.max(pred1)
    t1 = jnp.min(jnp.where(pred1 == v1, iota, big))
    m1 = iota == t1
    ev = jnp.exp(-pred)
    e0 = jnp.sum(jnp.where(m0, ev, 0.0))
    e1 = jnp.sum(jnp.where(m1, ev, 0.0))
    s = e0 + e1
    w0 = e0 / s
    w1 = e1 / s
    t0_ref[...] = jnp.full((1, 1), t0, jnp.int32)
    t1_ref[...] = jnp.full((1, 1), t1, jnp.int32)
    w0_ref[...] = jnp.full((1, 1), w0, jnp.float32)
    w1_ref[...] = jnp.full((1, 1), w1, jnp.float32)
    wrow = jnp.where(m0, w0, 0.0) + jnp.where(m1, w1, 0.0)
    bc_ref[...] = jnp.dot(wrow, be_ref[...],
                          preferred_element_type=jnp.float32)


def _route(predicate, Wp, bp, be):
    out_shapes = (
        jax.ShapeDtypeStruct((1, 1), jnp.int32),
        jax.ShapeDtypeStruct((1, 1), jnp.int32),
        jax.ShapeDtypeStruct((1, 1), jnp.float32),
        jax.ShapeDtypeStruct((1, 1), jnp.float32),
        jax.ShapeDtypeStruct((1, _D), jnp.float32),
    )
    return pl.pallas_call(
        _route_kernel,
        out_shape=out_shapes,
    )(predicate.reshape(1, _D), Wp, bp.reshape(1, _E), be)


def _moe_matmul_kernel(s_ref, x_ref, we0_ref, we1_ref, w0_ref, w1_ref,
                       bc_ref, o_ref, acc_ref):
    k = pl.program_id(2)
    nk = pl.num_programs(2)

    @pl.when(k == 0)
    def _():
        acc_ref[...] = jnp.zeros_like(acc_ref)

    wc = w0_ref[0, 0] * we0_ref[0] + w1_ref[0, 0] * we1_ref[0]
    acc_ref[...] += jax.lax.dot_general(
        x_ref[...], wc, (((1,), (1,)), ((), ())),
        preferred_element_type=jnp.float32)

    @pl.when(k == nk - 1)
    def _():
        o_ref[...] = acc_ref[...] + bc_ref[...]


def _moe_matmul(x, We, tops, w0, w1, bc):
    nm = _NTOK // _BM
    nn = _D // _BN
    nk = _D // _BK
    grid_spec = pltpu.PrefetchScalarGridSpec(
        num_scalar_prefetch=1,
        grid=(nm, nn, nk),
        in_specs=[
            pl.BlockSpec((_BM, _BK), lambda m, n, k, s: (m, k)),
            pl.BlockSpec((1, _BN, _BK), lambda m, n, k, s: (s[0], n, k)),
            pl.BlockSpec((1, _BN, _BK), lambda m, n, k, s: (s[1], n, k)),
            pl.BlockSpec((1, 1), lambda m, n, k, s: (0, 0)),
            pl.BlockSpec((1, 1), lambda m, n, k, s: (0, 0)),
            pl.BlockSpec((1, _BN), lambda m, n, k, s: (0, n)),
        ],
        out_specs=pl.BlockSpec((_BM, _BN), lambda m, n, k, s: (m, n)),
        scratch_shapes=[pltpu.VMEM((_BM, _BN), jnp.float32)],
    )
    return pl.pallas_call(
        _moe_matmul_kernel,
        grid_spec=grid_spec,
        out_shape=jax.ShapeDtypeStruct((_NTOK, _D), jnp.float32),
        compiler_params=pltpu.CompilerParams(
            dimension_semantics=("parallel", "parallel", "arbitrary"),
        ),
    )(tops, x, We, We, w0, w1, bc)


@functools.partial(jax.jit, static_argnums=())
def kernel(predicate, input, Wp, bp, We, be):
    t0, t1, w0, w1, bc = _route(predicate, Wp, bp, be)
    tops = jnp.concatenate([t0.reshape(1), t1.reshape(1)])
    return _moe_matmul(input, We, tops, w0, w1, bc)


# speedup vs baseline: 3.3166x; 1.6663x over previous
import functools

import jax
import jax.numpy as jnp
from jax.experimental import pallas as pl
from jax.experimental.pallas import tpu as pltpu

_E = 8
_D = 2048
_NTOK = 4096

_BM = 2048
_BN = 1024
_BK = 512


def _route_kernel(pred_ref, wp_ref, bp_ref, be_ref,
                  t0_ref, t1_ref, w0_ref, w1_ref, bc_ref):
    pred = jnp.dot(pred_ref[...], wp_ref[...],
                   preferred_element_type=jnp.float32) + bp_ref[...]
    iota = jax.lax.broadcasted_iota(jnp.int32, pred.shape, 1)
    big = jnp.int32(_E + 1)
    v0 = jnp.max(pred)
    t0 = jnp.min(jnp.where(pred == v0, iota, big))
    m0 = iota == t0
    pred1 = jnp.where(m0, -jnp.inf, pred)
    v1 = jnp.max(pred1)
    t1 = jnp.min(jnp.where(pred1 == v1, iota, big))
    m1 = iota == t1
    ev = jnp.exp(-pred)
    e0 = jnp.sum(jnp.where(m0, ev, 0.0))
    e1 = jnp.sum(jnp.where(m1, ev, 0.0))
    s = e0 + e1
    w0 = e0 / s
    w1 = e1 / s
    t0_ref[...] = jnp.full((1, 1), t0, jnp.int32)
    t1_ref[...] = jnp.full((1, 1), t1, jnp.int32)
    w0_ref[...] = jnp.full((1, 1), w0, jnp.float32)
    w1_ref[...] = jnp.full((1, 1), w1, jnp.float32)
    wrow = jnp.where(m0, w0, 0.0) + jnp.where(m1, w1, 0.0)
    bc_ref[...] = jnp.dot(wrow, be_ref[...],
                          preferred_element_type=jnp.float32)


def _route(predicate, Wp, bp, be):
    out_shapes = (
        jax.ShapeDtypeStruct((1, 1), jnp.int32),
        jax.ShapeDtypeStruct((1, 1), jnp.int32),
        jax.ShapeDtypeStruct((1, 1), jnp.float32),
        jax.ShapeDtypeStruct((1, 1), jnp.float32),
        jax.ShapeDtypeStruct((1, _D), jnp.float32),
    )
    return pl.pallas_call(
        _route_kernel,
        out_shape=out_shapes,
    )(predicate.reshape(1, _D), Wp, bp.reshape(1, _E), be)


def _moe_matmul_kernel(s_ref, x_ref, we0_ref, we1_ref, w0_ref, w1_ref,
                       bc_ref, o_ref, acc_ref):
    k = pl.program_id(2)
    nk = pl.num_programs(2)

    @pl.when(k == 0)
    def _():
        acc_ref[...] = jnp.zeros_like(acc_ref)

    wc = (w0_ref[0, 0] * we0_ref[0]
          + w1_ref[0, 0] * we1_ref[0]).astype(jnp.bfloat16)
    acc_ref[...] += jax.lax.dot_general(
        x_ref[...].astype(jnp.bfloat16), wc, (((1,), (1,)), ((), ())),
        preferred_element_type=jnp.float32)

    @pl.when(k == nk - 1)
    def _():
        o_ref[...] = acc_ref[...] + bc_ref[...]


def _moe_matmul(x, We, tops, w0, w1, bc):
    nm = _NTOK // _BM
    nn = _D // _BN
    nk = _D // _BK
    grid_spec = pltpu.PrefetchScalarGridSpec(
        num_scalar_prefetch=1,
        grid=(nm, nn, nk),
        in_specs=[
            pl.BlockSpec((_BM, _BK), lambda m, n, k, s: (m, k)),
            pl.BlockSpec((1, _BN, _BK), lambda m, n, k, s: (s[0], n, k)),
            pl.BlockSpec((1, _BN, _BK), lambda m, n, k, s: (s[1], n, k)),
            pl.BlockSpec((1, 1), lambda m, n, k, s: (0, 0)),
            pl.BlockSpec((1, 1), lambda m, n, k, s: (0, 0)),
            pl.BlockSpec((1, _BN), lambda m, n, k, s: (0, n)),
        ],
        out_specs=pl.BlockSpec((_BM, _BN), lambda m, n, k, s: (m, n)),
        scratch_shapes=[pltpu.VMEM((_BM, _BN), jnp.float32)],
    )
    return pl.pallas_call(
        _moe_matmul_kernel,
        grid_spec=grid_spec,
        out_shape=jax.ShapeDtypeStruct((_NTOK, _D), jnp.float32),
        compiler_params=pltpu.CompilerParams(
            dimension_semantics=("parallel", "parallel", "arbitrary"),
        ),
    )(tops, x, We, We, w0, w1, bc)


@functools.partial(jax.jit, static_argnums=())
def kernel(predicate, input, Wp, bp, We, be):
    t0, t1, w0, w1, bc = _route(predicate, Wp, bp, be)
    tops = jnp.concatenate([t0.reshape(1), t1.reshape(1)])
    return _moe_matmul(input, We, tops, w0, w1, bc)


# speedup vs baseline: 3.6015x; 1.0859x over previous
import functools

import jax
import jax.numpy as jnp
from jax.experimental import pallas as pl
from jax.experimental.pallas import tpu as pltpu

_E = 8
_D = 2048
_NTOK = 4096

_BKC = 256
_BM = 512


def _route_kernel(pred_ref, wp_ref, bp_ref, be_ref,
                  t0_ref, t1_ref, w0_ref, w1_ref, bc_ref):
    pred = jnp.dot(pred_ref[...], wp_ref[...],
                   preferred_element_type=jnp.float32) + bp_ref[...]
    iota = jax.lax.broadcasted_iota(jnp.int32, pred.shape, 1)
    big = jnp.int32(_E + 1)
    v0 = jnp.max(pred)
    t0 = jnp.min(jnp.where(pred == v0, iota, big))
    m0 = iota == t0
    pred1 = jnp.where(m0, -jnp.inf, pred)
    v1 = jnp.max(pred1)
    t1 = jnp.min(jnp.where(pred1 == v1, iota, big))
    m1 = iota == t1
    ev = jnp.exp(-pred)
    e0 = jnp.sum(jnp.where(m0, ev, 0.0))
    e1 = jnp.sum(jnp.where(m1, ev, 0.0))
    s = e0 + e1
    w0 = e0 / s
    w1 = e1 / s
    t0_ref[...] = jnp.full((1, 1), t0, jnp.int32)
    t1_ref[...] = jnp.full((1, 1), t1, jnp.int32)
    w0_ref[...] = jnp.full((1, 1), w0, jnp.float32)
    w1_ref[...] = jnp.full((1, 1), w1, jnp.float32)
    wrow = jnp.where(m0, w0, 0.0) + jnp.where(m1, w1, 0.0)
    bc_ref[...] = jnp.dot(wrow, be_ref[...],
                          preferred_element_type=jnp.float32)


def _route(predicate, Wp, bp, be):
    out_shapes = (
        jax.ShapeDtypeStruct((1, 1), jnp.int32),
        jax.ShapeDtypeStruct((1, 1), jnp.int32),
        jax.ShapeDtypeStruct((1, 1), jnp.float32),
        jax.ShapeDtypeStruct((1, 1), jnp.float32),
        jax.ShapeDtypeStruct((1, _D), jnp.float32),
    )
    return pl.pallas_call(
        _route_kernel,
        out_shape=out_shapes,
    )(predicate.reshape(1, _D), Wp, bp.reshape(1, _E), be)


def _combine_kernel(s_ref, we0_ref, we1_ref, w0_ref, w1_ref, wc_ref):
    wc_ref[...] = (w0_ref[0, 0] * we0_ref[0]
                   + w1_ref[0, 0] * we1_ref[0]).astype(jnp.bfloat16)


def _combine(We, tops, w0, w1):
    nk = _D // _BKC
    grid_spec = pltpu.PrefetchScalarGridSpec(
        num_scalar_prefetch=1,
        grid=(nk,),
        in_specs=[
            pl.BlockSpec((1, _D, _BKC), lambda k, s: (s[0], 0, k)),
            pl.BlockSpec((1, _D, _BKC), lambda k, s: (s[1], 0, k)),
            pl.BlockSpec((1, 1), lambda k, s: (0, 0)),
            pl.BlockSpec((1, 1), lambda k, s: (0, 0)),
        ],
        out_specs=pl.BlockSpec((_D, _BKC), lambda k, s: (0, k)),
    )
    return pl.pallas_call(
        _combine_kernel,
        grid_spec=grid_spec,
        out_shape=jax.ShapeDtypeStruct((_D, _D), jnp.bfloat16),
        compiler_params=pltpu.CompilerParams(
            dimension_semantics=("arbitrary",),
        ),
    )(tops, We, We, w0, w1)


def _matmul_kernel(x_ref, wc_ref, bc_ref, o_ref):
    xb = x_ref[...].astype(jnp.bfloat16)
    o_ref[...] = jax.lax.dot_general(
        xb, wc_ref[...], (((1,), (1,)), ((), ())),
        preferred_element_type=jnp.float32) + bc_ref[...]


def _matmul(x, Wc, bc):
    nm = _NTOK // _BM
    return pl.pallas_call(
        _matmul_kernel,
        grid=(nm,),
        in_specs=[
            pl.BlockSpec((_BM, _D), lambda m: (m, 0)),
            pl.BlockSpec((_D, _D), lambda m: (0, 0)),
            pl.BlockSpec((1, _D), lambda m: (0, 0)),
        ],
        out_specs=pl.BlockSpec((_BM, _D), lambda m: (m, 0)),
        out_shape=jax.ShapeDtypeStruct((_NTOK, _D), jnp.float32),
        compiler_params=pltpu.CompilerParams(
            dimension_semantics=("arbitrary",),
        ),
    )(x, Wc, bc)


@functools.partial(jax.jit, static_argnums=())
def kernel(predicate, input, Wp, bp, We, be):
    t0, t1, w0, w1, bc = _route(predicate, Wp, bp, be)
    tops = jnp.concatenate([t0.reshape(1), t1.reshape(1)])
    Wc = _combine(We, tops, w0, w1)
    return _matmul(input, Wc, bc)


# speedup vs baseline: 4.1476x; 1.1516x over previous
import functools

import jax
import jax.numpy as jnp
from jax.experimental import pallas as pl
from jax.experimental.pallas import tpu as pltpu

_E = 8
_D = 2048
_NTOK = 4096

_BKC = 256
_NKC = _D // _BKC
_BM = 512
_NM = _NTOK // _BM


def _route_kernel(pred_ref, wp_ref, bp_ref, be_ref,
                  tops_ref, w0_ref, w1_ref, bc_ref):
    pred = jnp.dot(pred_ref[...], wp_ref[...],
                   preferred_element_type=jnp.float32) + bp_ref[...]
    iota = jax.lax.broadcasted_iota(jnp.int32, pred.shape, 1)
    big = jnp.int32(_E + 1)
    v0 = jnp.max(pred)
    t0 = jnp.min(jnp.where(pred == v0, iota, big))
    m0 = iota == t0
    pred1 = jnp.where(m0, -jnp.inf, pred)
    v1 = jnp.max(pred1)
    t1 = jnp.min(jnp.where(pred1 == v1, iota, big))
    m1 = iota == t1
    ev = jnp.exp(-pred)
    e0 = jnp.sum(jnp.where(m0, ev, 0.0))
    e1 = jnp.sum(jnp.where(m1, ev, 0.0))
    s = e0 + e1
    w0 = e0 / s
    w1 = e1 / s
    iota2 = jax.lax.broadcasted_iota(jnp.int32, (1, 2), 1)
    tops_ref[...] = jnp.where(iota2 == 0, t0, t1)
    w0_ref[...] = jnp.full((1, 1), w0, jnp.float32)
    w1_ref[...] = jnp.full((1, 1), w1, jnp.float32)
    wrow = jnp.where(m0, w0, 0.0) + jnp.where(m1, w1, 0.0)
    bc_ref[...] = jnp.dot(wrow, be_ref[...],
                          preferred_element_type=jnp.float32)


def _route(predicate, Wp, bp, be):
    out_shapes = (
        jax.ShapeDtypeStruct((1, 2), jnp.int32),
        jax.ShapeDtypeStruct((1, 1), jnp.float32),
        jax.ShapeDtypeStruct((1, 1), jnp.float32),
        jax.ShapeDtypeStruct((1, _D), jnp.float32),
    )
    return pl.pallas_call(
        _route_kernel,
        out_shape=out_shapes,
    )(predicate.reshape(1, _D), Wp, bp.reshape(1, _E), be)


def _moe_kernel(s_ref, we0_ref, we1_ref, w0_ref, w1_ref, x_ref, bc_ref,
                o_ref, wct_ref):
    i = pl.program_id(0)

    @pl.when(i < _NKC)
    def _combine():
        wc = (w0_ref[0, 0] * we0_ref[0]
              + w1_ref[0, 0] * we1_ref[0]).astype(jnp.bfloat16)
        wct_ref[pl.ds(i * _BKC, _BKC), :] = wc.T

    @pl.when(i >= _NKC)
    def _matmul():
        xb = x_ref[...].astype(jnp.bfloat16)
        o_ref[...] = jax.lax.dot_general(
            xb, wct_ref[...], (((1,), (0,)), ((), ())),
            preferred_element_type=jnp.float32) + bc_ref[...]


def _moe(x, We, tops, w0, w1, bc):
    nkc = _NKC
    grid_spec = pltpu.PrefetchScalarGridSpec(
        num_scalar_prefetch=1,
        grid=(_NKC + _NM,),
        in_specs=[
            pl.BlockSpec((1, _D, _BKC),
                         lambda i, s: (s[0, 0], 0, jnp.minimum(i, nkc - 1))),
            pl.BlockSpec((1, _D, _BKC),
                         lambda i, s: (s[0, 1], 0, jnp.minimum(i, nkc - 1))),
            pl.BlockSpec((1, 1), lambda i, s: (0, 0)),
            pl.BlockSpec((1, 1), lambda i, s: (0, 0)),
            pl.BlockSpec((_BM, _D),
                         lambda i, s: (jnp.maximum(i - nkc, 0), 0)),
            pl.BlockSpec((1, _D), lambda i, s: (0, 0)),
        ],
        out_specs=pl.BlockSpec((_BM, _D),
                               lambda i, s: (jnp.maximum(i - nkc, 0), 0)),
        scratch_shapes=[pltpu.VMEM((_D, _D), jnp.bfloat16)],
    )
    return pl.pallas_call(
        _moe_kernel,
        grid_spec=grid_spec,
        out_shape=jax.ShapeDtypeStruct((_NTOK, _D), jnp.float32),
        compiler_params=pltpu.CompilerParams(
            dimension_semantics=("arbitrary",),
        ),
    )(tops, We, We, w0, w1, x, bc)


@functools.partial(jax.jit, static_argnums=())
def kernel(predicate, input, Wp, bp, We, be):
    tops, w0, w1, bc = _route(predicate, Wp, bp, be)
    return _moe(input, We, tops, w0, w1, bc)
